# Initial kernel scaffold; baseline (speedup 1.0000x reference)
import functools

import jax
import jax.numpy as jnp
from jax import lax
from jax.experimental import pallas as pl
from jax.experimental.pallas import tpu as pltpu
from jax.experimental.pallas import tpu_sc as plsc

VOCAB = 201
HIDDEN = 1024
OUT = 41
OUT_PAD = 48
B_TOK = 4 * 4096

_NC, _NS = 2, 16
_NW = _NC * _NS
_BPW = B_TOK // _NW
_CHUNK = 128
_NCHUNK = _BPW // _CHUNK


def _proj_body(emb_ref, wt_ref, b_ref, p_ref):
    p_ref[...] = (
        jnp.dot(emb_ref[...], wt_ref[...], preferred_element_type=jnp.float32)
        + b_ref[...]
    )


def _project_table(emb_table, wt_pad, b_pad):
    return pl.pallas_call(
        _proj_body,
        out_shape=jax.ShapeDtypeStruct((VOCAB, OUT_PAD), jnp.float32),
    )(emb_table, wt_pad, b_pad)


_sc_mesh = plsc.VectorSubcoreMesh(core_axis_name="c", subcore_axis_name="s")


@functools.partial(
    pl.kernel,
    out_type=jax.ShapeDtypeStruct((B_TOK, OUT_PAD), jnp.float32),
    mesh=_sc_mesh,
    scratch_types=[
        pltpu.VMEM((_NCHUNK, _CHUNK), jnp.int32),
        pltpu.VMEM((_BPW, OUT_PAD), jnp.float32),
        pltpu.SemaphoreType.DMA,
    ],
)
def _sc_gather(table_hbm, idx_hbm, out_hbm, idx_v, rows_v, sem):
    wid = lax.axis_index("s") * _NC + lax.axis_index("c")
    pltpu.sync_copy(idx_hbm.at[wid], idx_v)
    copies = []
    for j in range(_NCHUNK):
        copies.append(
            pltpu.async_copy(
                table_hbm.at[idx_v.at[j]],
                rows_v.at[pl.ds(j * _CHUNK, _CHUNK)],
                sem,
            )
        )
    for c in copies:
        c.wait()
    pltpu.sync_copy(rows_v, out_hbm.at[pl.ds(wid * _BPW, _BPW)])


def kernel(x, emb_table, W, b):
    wt_pad = jnp.zeros((HIDDEN, OUT_PAD), jnp.float32).at[:, :OUT].set(W.T)
    b_pad = jnp.zeros((1, OUT_PAD), jnp.float32).at[:, :OUT].set(b)
    table = _project_table(emb_table, wt_pad, b_pad)
    idx = x.reshape(_NW, _NCHUNK, _CHUNK)
    out48 = _sc_gather(table, idx)
    return out48[:, :OUT].reshape(x.shape[0], x.shape[1], OUT)


# speedup vs baseline: 1.8428x; 1.8428x over previous
import functools

import jax
import jax.numpy as jnp
from jax import lax
from jax.experimental import pallas as pl
from jax.experimental.pallas import tpu as pltpu
from jax.experimental.pallas import tpu_sc as plsc

VOCAB = 201
HIDDEN = 1024
OUT = 41
OUT_PAD = 128
B_TOK = 4 * 4096

_NC, _NS = 2, 16
_NW = _NC * _NS
_BPW = B_TOK // _NW
_CHUNK = 128
_NCHUNK = _BPW // _CHUNK


def _proj_body(emb_ref, wt_ref, b_ref, p_ref):
    p_ref[...] = (
        jnp.dot(emb_ref[...], wt_ref[...], preferred_element_type=jnp.float32)
        + b_ref[...]
    )


def _project_table(emb_table, wt_pad, b_pad):
    return pl.pallas_call(
        _proj_body,
        out_shape=jax.ShapeDtypeStruct((VOCAB, OUT_PAD), jnp.float32),
    )(emb_table, wt_pad, b_pad)


@functools.cache
def _make_sc_gather():
    mesh = plsc.VectorSubcoreMesh(core_axis_name="c", subcore_axis_name="s")

    @functools.partial(
        pl.kernel,
        out_type=jax.ShapeDtypeStruct((B_TOK, OUT_PAD), jnp.float32),
        mesh=mesh,
        scratch_types=[
            pltpu.VMEM((_NCHUNK, _CHUNK), jnp.int32),
            pltpu.VMEM((_BPW, OUT_PAD), jnp.float32),
            pltpu.SemaphoreType.DMA,
        ],
    )
    def _sc_gather(table_hbm, idx_hbm, out_hbm, idx_v, rows_v, sem):
        wid = lax.axis_index("s") * _NC + lax.axis_index("c")
        pltpu.sync_copy(idx_hbm.at[wid], idx_v)
        copies = []
        for j in range(_NCHUNK):
            copies.append(
                pltpu.async_copy(
                    table_hbm.at[idx_v.at[j]],
                    rows_v.at[pl.ds(j * _CHUNK, _CHUNK)],
                    sem,
                )
            )
        for c in copies:
            c.wait()
        pltpu.sync_copy(rows_v, out_hbm.at[pl.ds(wid * _BPW, _BPW)])

    return _sc_gather


def kernel(x, emb_table, W, b):
    wt_pad = jnp.zeros((HIDDEN, OUT_PAD), jnp.float32).at[:, :OUT].set(W.T)
    b_pad = jnp.zeros((1, OUT_PAD), jnp.float32).at[:, :OUT].set(b)
    table = _project_table(emb_table, wt_pad, b_pad)
    idx = x.reshape(_NW, _NCHUNK, _CHUNK)
    out48 = _make_sc_gather()(table, idx)
    return out48[:, :OUT].reshape(x.shape[0], x.shape[1], OUT)
